# Initial kernel scaffold; baseline (speedup 1.0000x reference)
import jax
import jax.numpy as jnp
from jax.experimental import pallas as pl


def kernel(x, edge_index, w, W1, b1, W2, b2, Wc, bc):
    raise NotImplementedError("write your pallas kernel here")


# speedup vs baseline: 5.9651x; 5.9651x over previous
import functools
import jax
import jax.numpy as jnp
from jax import lax
from jax.experimental import pallas as pl
from jax.experimental.pallas import tpu as pltpu
from jax.experimental.pallas import tpu_sc as plsc

F32 = jnp.float32
I32 = jnp.int32

NC = 2
NS = 16
NW = NC * NS
LANES = 16
CHUNK = 128
N = 10000
N_PAD = 10240
D = 128
ROWS_PER_TILE = N_PAD // NS

def _mesh():
  return plsc.VectorSubcoreMesh(
      core_axis_name="c", subcore_axis_name="s", num_cores=NC, num_subcores=NS)


def _worker_ids():
  c = lax.axis_index("c")
  s = lax.axis_index("s")
  return c, s, c * NS + s


def _deg_body(cpw, src_hbm, w_hbm, out_hbm, idx_v, val_v, acc_sh, zset):
  c, s, w = _worker_ids()
  def zloop(i, _):
    val_v[pl.ds(i * LANES, LANES)] = jnp.zeros((LANES,), F32)
    return 0
  lax.fori_loop(0, CHUNK // LANES, zloop, 0)
  nz = ROWS_PER_TILE // CHUNK
  for r in range(nz):
    pltpu.sync_copy(val_v, acc_sh.at[pl.ds(s * ROWS_PER_TILE + r * CHUNK, CHUNK)])
  plsc.subcore_barrier()

  base = w * cpw * CHUNK
  def chunk(k, _):
    off = base + k * CHUNK
    pltpu.sync_copy(src_hbm.at[pl.ds(off, CHUNK)], idx_v)
    pltpu.sync_copy(w_hbm.at[pl.ds(off, CHUNK)], val_v)
    pltpu.sync_copy(val_v, acc_sh.at[idx_v], add=True)
    return 0
  lax.fori_loop(0, cpw, chunk, 0)
  plsc.subcore_barrier()
  pltpu.sync_copy(acc_sh.at[pl.ds(s * ROWS_PER_TILE, ROWS_PER_TILE)],
                  out_hbm.at[c, pl.ds(s * ROWS_PER_TILE, ROWS_PER_TILE)])


def _sc_deg(src_p, w_p, cpw):
  k = pl.kernel(
      functools.partial(_deg_body, cpw),
      out_type=jax.ShapeDtypeStruct((NC, N_PAD), F32),
      mesh=_mesh(),
      compiler_params=pltpu.CompilerParams(needs_layout_passes=False, use_tc_tiling_on_sc=False),
      scratch_types=[
          pltpu.VMEM((CHUNK,), I32),
          pltpu.VMEM((CHUNK,), F32),
          pltpu.VMEM_SHARED((N_PAD,), F32),
          pltpu.SemaphoreType.DMA,
      ],
  )
  return k(src_p, w_p)


def _normw_body(cpw, dis_hbm, src_hbm, dst_hbm, w_hbm, out_hbm,
                dis_v, src_v, dst_v, w_v, o_v):
  c, s, w = _worker_ids()
  pltpu.sync_copy(dis_hbm, dis_v)
  base = w * cpw * CHUNK
  def chunk(k, _):
    off = base + k * CHUNK
    pltpu.sync_copy(src_hbm.at[pl.ds(off, CHUNK)], src_v)
    pltpu.sync_copy(dst_hbm.at[pl.ds(off, CHUNK)], dst_v)
    pltpu.sync_copy(w_hbm.at[pl.ds(off, CHUNK)], w_v)
    for g in range(CHUNK // LANES):
      sl = pl.ds(g * LANES, LANES)
      a = plsc.load_gather(dis_v, [src_v[sl]])
      b = plsc.load_gather(dis_v, [dst_v[sl]])
      o_v[sl] = -(a * w_v[sl] * b)
    pltpu.sync_copy(o_v, out_hbm.at[pl.ds(off, CHUNK)])
    return 0
  lax.fori_loop(0, cpw, chunk, 0)


def _sc_normw(dis, src_p, dst_p, w_p, cpw, e_pad):
  k = pl.kernel(
      functools.partial(_normw_body, cpw),
      out_type=jax.ShapeDtypeStruct((e_pad,), F32),
      mesh=_mesh(),
      compiler_params=pltpu.CompilerParams(needs_layout_passes=False, use_tc_tiling_on_sc=False),
      scratch_types=[
          pltpu.VMEM((N_PAD,), F32),
          pltpu.VMEM((CHUNK,), I32),
          pltpu.VMEM((CHUNK,), I32),
          pltpu.VMEM((CHUNK,), F32),
          pltpu.VMEM((CHUNK,), F32),
      ],
  )
  return k(dis, src_p, dst_p, w_p)


def _spmv_body(cpw, tab_hbm, src_hbm, dst_hbm, nw_hbm, out_hbm,
               src_v, dst_v, nw_v, rows_v, acc_sh, sem):
  c, s, w = _worker_ids()
  def zloop(i, _):
    for cc in range(D // LANES):
      rows_v[i, pl.ds(cc * LANES, LANES)] = jnp.zeros((LANES,), F32)
    return 0
  lax.fori_loop(0, CHUNK, zloop, 0)
  for r in range(ROWS_PER_TILE // CHUNK):
    pltpu.sync_copy(rows_v, acc_sh.at[pl.ds(s * ROWS_PER_TILE + r * CHUNK, CHUNK)])
  plsc.subcore_barrier()

  base = w * cpw * CHUNK
  def chunk(k, _):
    off = base + k * CHUNK
    pltpu.sync_copy(src_hbm.at[pl.ds(off, CHUNK)], src_v)
    pltpu.sync_copy(dst_hbm.at[pl.ds(off, CHUNK)], dst_v)
    pltpu.sync_copy(nw_hbm.at[pl.ds(off, CHUNK)], nw_v)
    pltpu.async_copy(tab_hbm.at[src_v], rows_v, sem).wait()
    def scale(e, _):
      sp = plsc.load_gather(nw_v, [jnp.full((LANES,), e, I32)])
      for cc in range(D // LANES):
        sl = pl.ds(cc * LANES, LANES)
        rows_v[e, sl] = rows_v[e, sl] * sp
      return 0
    lax.fori_loop(0, CHUNK, scale, 0)
    pltpu.sync_copy(rows_v, acc_sh.at[dst_v], add=True)
    return 0
  lax.fori_loop(0, cpw, chunk, 0)
  plsc.subcore_barrier()
  pltpu.sync_copy(acc_sh.at[pl.ds(s * ROWS_PER_TILE, ROWS_PER_TILE)],
                  out_hbm.at[c, pl.ds(s * ROWS_PER_TILE, ROWS_PER_TILE)])


def _sc_spmv(tab, src_p, dst_p, nw_p, cpw):
  k = pl.kernel(
      functools.partial(_spmv_body, cpw),
      out_type=jax.ShapeDtypeStruct((NC, N_PAD, D), F32),
      mesh=_mesh(),
      compiler_params=pltpu.CompilerParams(needs_layout_passes=False, use_tc_tiling_on_sc=False),
      scratch_types=[
          pltpu.VMEM((CHUNK,), I32),
          pltpu.VMEM((CHUNK,), I32),
          pltpu.VMEM((CHUNK,), F32),
          pltpu.VMEM((CHUNK, D), F32),
          pltpu.VMEM_SHARED((N_PAD, D), F32),
          pltpu.SemaphoreType.DMA,
      ],
  )
  return k(tab, src_p, dst_p, nw_p)


def _edge_body(cpw, t4_hbm, src_hbm, dst_hbm, o0_hbm, o1_hbm,
               t4_v, src_v, dst_v, o0_v, o1_v):
  c, s, w = _worker_ids()
  pltpu.sync_copy(t4_hbm, t4_v)
  k0 = jnp.zeros((LANES,), I32)
  k1 = jnp.full((LANES,), 1, I32)
  k2 = jnp.full((LANES,), 2, I32)
  k3 = jnp.full((LANES,), 3, I32)
  base = w * cpw * CHUNK
  def chunk(k, _):
    off = base + k * CHUNK
    pltpu.sync_copy(src_hbm.at[pl.ds(off, CHUNK)], src_v)
    pltpu.sync_copy(dst_hbm.at[pl.ds(off, CHUNK)], dst_v)
    for g in range(CHUNK // LANES):
      sl = pl.ds(g * LANES, LANES)
      isrc = src_v[sl]
      idst = dst_v[sl]
      o0_v[sl] = (plsc.load_gather(t4_v, [isrc, k0]) +
                  plsc.load_gather(t4_v, [idst, k2]))
      o1_v[sl] = (plsc.load_gather(t4_v, [isrc, k1]) +
                  plsc.load_gather(t4_v, [idst, k3]))
    pltpu.sync_copy(o0_v, o0_hbm.at[pl.ds(off, CHUNK)])
    pltpu.sync_copy(o1_v, o1_hbm.at[pl.ds(off, CHUNK)])
    return 0
  lax.fori_loop(0, cpw, chunk, 0)


def _sc_edge_out(t4, src_p, dst_p, cpw, e_pad):
  k = pl.kernel(
      functools.partial(_edge_body, cpw),
      out_type=(jax.ShapeDtypeStruct((e_pad,), F32),
                jax.ShapeDtypeStruct((e_pad,), F32)),
      mesh=_mesh(),
      compiler_params=pltpu.CompilerParams(needs_layout_passes=False, use_tc_tiling_on_sc=False),
      scratch_types=[
          pltpu.VMEM((N_PAD, 4), F32),
          pltpu.VMEM((CHUNK,), I32),
          pltpu.VMEM((CHUNK,), I32),
          pltpu.VMEM((CHUNK,), F32),
          pltpu.VMEM((CHUNK,), F32),
      ],
  )
  return k(t4, src_p, dst_p)


def _dis_kernel_body(deg_ref, out_ref):
  d = deg_ref[0] + deg_ref[1]
  out_ref[...] = jnp.where(d > 0, lax.rsqrt(jnp.where(d > 0, d, 1.0)), 0.0)


def _tc_dis(deg2):
  deg2 = deg2.reshape(NC, N_PAD // D, D)
  out = pl.pallas_call(
      _dis_kernel_body,
      out_shape=jax.ShapeDtypeStruct((N_PAD // D, D), F32),
  )(deg2)
  return out.reshape(N_PAD)


_ROW_BLK = 1024


def _psum_body(p_ref, o_ref):
  o_ref[...] = p_ref[0] + p_ref[1]


def _tc_psum(P):
  grid = (N_PAD // _ROW_BLK,)
  return pl.pallas_call(
      _psum_body,
      grid=grid,
      in_specs=[pl.BlockSpec((NC, _ROW_BLK, D), lambda i: (0, i, 0))],
      out_specs=pl.BlockSpec((_ROW_BLK, D), lambda i: (i, 0)),
      out_shape=jax.ShapeDtypeStruct((N_PAD, D), F32),
  )(P)


def _layer_body(relu, x_ref, t1_ref, q_ref, w_ref, b_ref, o_ref):
  x = x_ref[...]
  t1 = t1_ref[...]
  t2 = 2.0 * (q_ref[0] + q_ref[1]) - x
  o = (jnp.dot(x, w_ref[0], preferred_element_type=F32) +
       jnp.dot(t1, w_ref[1], preferred_element_type=F32) +
       jnp.dot(t2, w_ref[2], preferred_element_type=F32) + b_ref[...])
  if relu:
    o = jnp.maximum(o, 0.0)
  o_ref[...] = o


def _tc_layer(x_pad, t1, Q, W, b, relu):
  grid = (N_PAD // _ROW_BLK,)
  return pl.pallas_call(
      functools.partial(_layer_body, relu),
      grid=grid,
      in_specs=[
          pl.BlockSpec((_ROW_BLK, D), lambda i: (i, 0)),
          pl.BlockSpec((_ROW_BLK, D), lambda i: (i, 0)),
          pl.BlockSpec((NC, _ROW_BLK, D), lambda i: (0, i, 0)),
          pl.BlockSpec((3, D, D), lambda i: (0, 0, 0)),
          pl.BlockSpec((1, D), lambda i: (0, 0)),
      ],
      out_specs=pl.BlockSpec((_ROW_BLK, D), lambda i: (i, 0)),
      out_shape=jax.ShapeDtypeStruct((N_PAD, D), F32),
  )(x_pad, t1, Q, W, b.reshape(1, D))


def _head_body(h_ref, w_ref, b_ref, o_ref):
  o_ref[...] = (jnp.dot(h_ref[...], w_ref[...], preferred_element_type=F32)
                + b_ref[...])


def _tc_head(h, wcat, bvec):
  grid = (N_PAD // _ROW_BLK,)
  return pl.pallas_call(
      _head_body,
      grid=grid,
      in_specs=[
          pl.BlockSpec((_ROW_BLK, D), lambda i: (i, 0)),
          pl.BlockSpec((D, D), lambda i: (0, 0)),
          pl.BlockSpec((1, D), lambda i: (0, 0)),
      ],
      out_specs=pl.BlockSpec((_ROW_BLK, D), lambda i: (i, 0)),
      out_shape=jax.ShapeDtypeStruct((N_PAD, D), F32),
  )(h, wcat, bvec)


def kernel(x, edge_index, w, W1, b1, W2, b2, Wc, bc):
  n, d = x.shape
  e = w.shape[0]
  src = edge_index[0].astype(I32)
  dst = edge_index[1].astype(I32)

  grain = NW * CHUNK
  e_pad = ((e + grain - 1) // grain) * grain
  cpw = e_pad // grain
  pads = e_pad - e
  pad_idx = (n + (jnp.arange(pads, dtype=I32) % (N_PAD - n))).astype(I32)
  src_p = jnp.concatenate([src, pad_idx])
  dst_p = jnp.concatenate([dst, pad_idx])
  w_p = jnp.concatenate([w, jnp.zeros((pads,), F32)])

  x_pad = jnp.pad(x, ((0, N_PAD - n), (0, 0)))

  deg2 = _sc_deg(src_p, w_p, cpw)
  dis = _tc_dis(deg2)
  nw_p = _sc_normw(dis, src_p, dst_p, w_p, cpw, e_pad)

  P1 = _sc_spmv(x_pad, src_p, dst_p, nw_p, cpw)
  t1 = _tc_psum(P1)
  Q1 = _sc_spmv(t1, src_p, dst_p, nw_p, cpw)
  h1 = _tc_layer(x_pad, t1, Q1, W1, b1, relu=True)

  P2 = _sc_spmv(h1, src_p, dst_p, nw_p, cpw)
  t2 = _tc_psum(P2)
  Q2 = _sc_spmv(t2, src_p, dst_p, nw_p, cpw)
  h2 = _tc_layer(h1, t2, Q2, W2, b2, relu=False)

  wcat = jnp.zeros((D, D), F32).at[:, 0:2].set(Wc[:d]).at[:, 2:4].set(Wc[d:])
  bvec = jnp.zeros((1, D), F32).at[0, 2:4].set(bc)
  t4 = _tc_head(h2, wcat, bvec)[:, :4]
  o0, o1 = _sc_edge_out(t4, src_p, dst_p, cpw, e_pad)
  return jnp.stack([o0[:e], o1[:e]], axis=-1)


# speedup vs baseline: 13.4095x; 2.2480x over previous
import functools
import jax
import jax.numpy as jnp
from jax import lax
from jax.experimental import pallas as pl
from jax.experimental.pallas import tpu as pltpu
from jax.experimental.pallas import tpu_sc as plsc

F32 = jnp.float32
I32 = jnp.int32

NC = 2
NS = 16
NW = NC * NS
LANES = 16
CHUNK = 128
N = 10000
N_PAD = 10240
D = 128
ROWS_PER_TILE = N_PAD // NS
NBUF = 2
PKBITS = 14

_params = pltpu.CompilerParams(
    needs_layout_passes=False, use_tc_tiling_on_sc=False)


def _mesh():
  return plsc.VectorSubcoreMesh(
      core_axis_name="c", subcore_axis_name="s", num_cores=NC, num_subcores=NS)


def _worker_ids():
  c = lax.axis_index("c")
  s = lax.axis_index("s")
  return c, s, c * NS + s


def _zero_vmem_1d(ref, nwords):
  def zloop(i, _):
    ref[pl.ds(i * LANES, LANES)] = jnp.zeros((LANES,), F32)
    return 0
  lax.fori_loop(0, nwords // LANES, zloop, 0)


def _deg_body(cpw, src3_hbm, w3_hbm, out_hbm, src_all, w_all, zbuf, acc_sh,
              sem0, sem1):
  c, s, w = _worker_ids()
  _zero_vmem_1d(zbuf, CHUNK)
  for r in range(ROWS_PER_TILE // CHUNK):
    pltpu.sync_copy(zbuf, acc_sh.at[pl.ds(s * ROWS_PER_TILE + r * CHUNK, CHUNK)])
  pltpu.sync_copy(src3_hbm.at[w], src_all)
  pltpu.sync_copy(w3_hbm.at[w], w_all)
  plsc.subcore_barrier()

  sems = (sem0, sem1)
  waves = cpw // LANES
  for wave in range(waves):
    for i in range(LANES):
      j = wave * LANES + i
      pltpu.async_copy(w_all.at[j], acc_sh.at[src_all.at[j]], sems[wave % 2],
                       add=True)
    if wave > 0:
      for i in range(LANES):
        jj = (wave - 1) * LANES + i
        pltpu.make_async_copy(w_all.at[jj], acc_sh.at[src_all.at[jj]],
                              sems[(wave - 1) % 2]).wait()
  for i in range(LANES):
    jj = (waves - 1) * LANES + i
    pltpu.make_async_copy(w_all.at[jj], acc_sh.at[src_all.at[jj]],
                          sems[(waves - 1) % 2]).wait()
  plsc.subcore_barrier()
  pltpu.sync_copy(acc_sh.at[pl.ds(s * ROWS_PER_TILE, ROWS_PER_TILE)],
                  out_hbm.at[c, pl.ds(s * ROWS_PER_TILE, ROWS_PER_TILE)])


def _sc_deg(src3, w3, cpw):
  k = pl.kernel(
      functools.partial(_deg_body, cpw),
      out_type=jax.ShapeDtypeStruct((NC, N_PAD), F32),
      mesh=_mesh(),
      compiler_params=_params,
      scratch_types=[
          pltpu.VMEM((cpw, CHUNK), I32),
          pltpu.VMEM((cpw, CHUNK), F32),
          pltpu.VMEM((CHUNK,), F32),
          pltpu.VMEM_SHARED((N_PAD,), F32),
          pltpu.SemaphoreType.DMA,
          pltpu.SemaphoreType.DMA,
      ],
  )
  return k(src3, w3)


def _spmv_body(cpw, tab_hbm, pk3_hbm, w3_hbm, out_hbm,
               pk_all, srcb, dstb, wb, rows, acc_sh, gsems, wsems):
  c, s, w = _worker_ids()
  def zrow(i, _):
    for cc in range(D // LANES):
      rows[0][i, pl.ds(cc * LANES, LANES)] = jnp.zeros((LANES,), F32)
    return 0
  lax.fori_loop(0, CHUNK, zrow, 0)
  for r in range(ROWS_PER_TILE // CHUNK):
    pltpu.sync_copy(rows[0], acc_sh.at[pl.ds(s * ROWS_PER_TILE + r * CHUNK, CHUNK)])
  pltpu.sync_copy(pk3_hbm.at[w], pk_all)
  plsc.subcore_barrier()

  mask = jnp.full((LANES,), (1 << PKBITS) - 1, I32)
  shift = jnp.full((LANES,), PKBITS, I32)

  def unpack(j, b):
    for g in range(CHUNK // LANES):
      sl = pl.ds(g * LANES, LANES)
      p = pk_all[j, sl]
      srcb[b][sl] = p & mask
      dstb[b][sl] = lax.shift_right_logical(p, shift)

  def issue_gather(j, b):
    pltpu.async_copy(tab_hbm.at[srcb[b]], rows[b], gsems[b])
    pltpu.async_copy(w3_hbm.at[w, j], wb[b], gsems[b])

  def wait_gather(j, b):
    pltpu.make_async_copy(tab_hbm.at[srcb[b]], rows[b], gsems[b]).wait()
    pltpu.make_async_copy(w3_hbm.at[w, j], wb[b], gsems[b]).wait()

  unpack(0, 0)
  issue_gather(0, 0)

  def body(j0, _):
    for b in range(NBUF):
      j = j0 * NBUF + b
      bn = (b + 1) % NBUF

      @pl.when(j + 1 < cpw)
      def _():
        @pl.when(j >= 1)
        def _():
          pltpu.make_async_copy(rows[bn], acc_sh.at[dstb[bn]],
                                wsems[bn]).wait()
        unpack(j + 1, bn)
        issue_gather(j + 1, bn)

      wait_gather(j, b)
      def scale(e, _):
        sp = plsc.load_gather(wb[b], [jnp.full((LANES,), e, I32)])
        for cc in range(D // LANES):
          sl = pl.ds(cc * LANES, LANES)
          rows[b][e, sl] = rows[b][e, sl] * sp
        return 0
      lax.fori_loop(0, CHUNK, scale, 0)
      pltpu.async_copy(rows[b], acc_sh.at[dstb[b]], wsems[b], add=True)
    return 0
  lax.fori_loop(0, cpw // NBUF, body, 0)
  for b in range(NBUF):
    pltpu.make_async_copy(rows[b], acc_sh.at[dstb[b]], wsems[b]).wait()
  plsc.subcore_barrier()
  pltpu.sync_copy(acc_sh.at[pl.ds(s * ROWS_PER_TILE, ROWS_PER_TILE)],
                  out_hbm.at[c, pl.ds(s * ROWS_PER_TILE, ROWS_PER_TILE)])


def _sc_spmv(tab, pk3, w3, cpw):
  k = pl.kernel(
      functools.partial(_spmv_body, cpw),
      out_type=jax.ShapeDtypeStruct((NC, N_PAD, D), F32),
      mesh=_mesh(),
      compiler_params=_params,
      scratch_types=[
          pltpu.VMEM((cpw, CHUNK), I32),
          [pltpu.VMEM((CHUNK,), I32) for _ in range(NBUF)],
          [pltpu.VMEM((CHUNK,), I32) for _ in range(NBUF)],
          [pltpu.VMEM((CHUNK,), F32) for _ in range(NBUF)],
          [pltpu.VMEM((CHUNK, D), F32) for _ in range(NBUF)],
          pltpu.VMEM_SHARED((N_PAD, D), F32),
          [pltpu.SemaphoreType.DMA for _ in range(NBUF)],
          [pltpu.SemaphoreType.DMA for _ in range(NBUF)],
      ],
  )
  return k(tab, pk3, w3)


def _edge_body(cpw, t4_hbm, src3_hbm, dst3_hbm, o0_hbm, o1_hbm,
               t4_v, src_all, dst_all, o0_all, o1_all):
  c, s, w = _worker_ids()
  pltpu.sync_copy(t4_hbm, t4_v)
  pltpu.sync_copy(src3_hbm.at[w], src_all)
  pltpu.sync_copy(dst3_hbm.at[w], dst_all)
  k0 = jnp.zeros((LANES,), I32)
  k1 = jnp.full((LANES,), 1, I32)
  k2 = jnp.full((LANES,), 2, I32)
  k3 = jnp.full((LANES,), 3, I32)

  def chunk(j, _):
    for g in range(CHUNK // LANES):
      sl = pl.ds(g * LANES, LANES)
      isrc = src_all[j, sl]
      idst = dst_all[j, sl]
      o0_all[j, sl] = (plsc.load_gather(t4_v, [isrc, k0]) +
                       plsc.load_gather(t4_v, [idst, k2]))
      o1_all[j, sl] = (plsc.load_gather(t4_v, [isrc, k1]) +
                       plsc.load_gather(t4_v, [idst, k3]))
    return 0
  lax.fori_loop(0, cpw, chunk, 0)
  pltpu.sync_copy(o0_all, o0_hbm.at[w])
  pltpu.sync_copy(o1_all, o1_hbm.at[w])


def _sc_edge_out(t4, src3, dst3, cpw):
  k = pl.kernel(
      functools.partial(_edge_body, cpw),
      out_type=(jax.ShapeDtypeStruct((NW, cpw, CHUNK), F32),
                jax.ShapeDtypeStruct((NW, cpw, CHUNK), F32)),
      mesh=_mesh(),
      compiler_params=_params,
      scratch_types=[
          pltpu.VMEM((N_PAD, 4), F32),
          pltpu.VMEM((cpw, CHUNK), I32),
          pltpu.VMEM((cpw, CHUNK), I32),
          pltpu.VMEM((cpw, CHUNK), F32),
          pltpu.VMEM((cpw, CHUNK), F32),
      ],
  )
  return k(t4, src3, dst3)


def _dis_kernel_body(deg_ref, out_ref):
  d = deg_ref[0] + deg_ref[1]
  out_ref[...] = jnp.where(d > 0, lax.rsqrt(jnp.where(d > 0, d, 1.0)), 0.0)


def _tc_dis(deg2):
  deg2 = deg2.reshape(NC, N_PAD // D, D)
  return pl.pallas_call(
      _dis_kernel_body,
      out_shape=jax.ShapeDtypeStruct((N_PAD // D, D), F32),
  )(deg2)


_ROW_BLK = 1024


def _scale_body(x_ref, d_ref, o_ref):
  o_ref[...] = x_ref[...] * d_ref[...]


def _tc_scale(x_pad, disc):
  grid = (N_PAD // _ROW_BLK,)
  return pl.pallas_call(
      _scale_body,
      grid=grid,
      in_specs=[
          pl.BlockSpec((_ROW_BLK, D), lambda i: (i, 0)),
          pl.BlockSpec((_ROW_BLK, 1), lambda i: (i, 0)),
      ],
      out_specs=pl.BlockSpec((_ROW_BLK, D), lambda i: (i, 0)),
      out_shape=jax.ShapeDtypeStruct((N_PAD, D), F32),
  )(x_pad, disc)


def _psum2_body(p_ref, d_ref, t_ref, ts_ref):
  dcol = d_ref[...]
  t = -(p_ref[0] + p_ref[1]) * dcol
  t_ref[...] = t
  ts_ref[...] = t * dcol


def _tc_psum2(P, disc):
  grid = (N_PAD // _ROW_BLK,)
  return pl.pallas_call(
      _psum2_body,
      grid=grid,
      in_specs=[
          pl.BlockSpec((NC, _ROW_BLK, D), lambda i: (0, i, 0)),
          pl.BlockSpec((_ROW_BLK, 1), lambda i: (i, 0)),
      ],
      out_specs=(pl.BlockSpec((_ROW_BLK, D), lambda i: (i, 0)),
                 pl.BlockSpec((_ROW_BLK, D), lambda i: (i, 0))),
      out_shape=(jax.ShapeDtypeStruct((N_PAD, D), F32),
                 jax.ShapeDtypeStruct((N_PAD, D), F32)),
  )(P, disc)


def _layer_body(relu, x_ref, t1_ref, q_ref, d_ref, w_ref, b_ref,
                o_ref, os_ref):
  x = x_ref[...]
  t1 = t1_ref[...]
  dcol = d_ref[...]
  t2 = -2.0 * (q_ref[0] + q_ref[1]) * dcol - x
  o = (jnp.dot(x, w_ref[0], preferred_element_type=F32) +
       jnp.dot(t1, w_ref[1], preferred_element_type=F32) +
       jnp.dot(t2, w_ref[2], preferred_element_type=F32) + b_ref[...])
  if relu:
    o = jnp.maximum(o, 0.0)
  o_ref[...] = o
  os_ref[...] = o * dcol


def _tc_layer(x_pad, t1, Q, disc, W, b, relu):
  grid = (N_PAD // _ROW_BLK,)
  return pl.pallas_call(
      functools.partial(_layer_body, relu),
      grid=grid,
      in_specs=[
          pl.BlockSpec((_ROW_BLK, D), lambda i: (i, 0)),
          pl.BlockSpec((_ROW_BLK, D), lambda i: (i, 0)),
          pl.BlockSpec((NC, _ROW_BLK, D), lambda i: (0, i, 0)),
          pl.BlockSpec((_ROW_BLK, 1), lambda i: (i, 0)),
          pl.BlockSpec((3, D, D), lambda i: (0, 0, 0)),
          pl.BlockSpec((1, D), lambda i: (0, 0)),
      ],
      out_specs=(pl.BlockSpec((_ROW_BLK, D), lambda i: (i, 0)),
                 pl.BlockSpec((_ROW_BLK, D), lambda i: (i, 0))),
      out_shape=(jax.ShapeDtypeStruct((N_PAD, D), F32),
                 jax.ShapeDtypeStruct((N_PAD, D), F32)),
  )(x_pad, t1, Q, disc, W, b.reshape(1, D))


def _head_body(h_ref, w_ref, b_ref, o_ref):
  o_ref[...] = (jnp.dot(h_ref[...], w_ref[...], preferred_element_type=F32)
                + b_ref[...])


def _tc_head(h, wcat, bvec):
  grid = (N_PAD // _ROW_BLK,)
  return pl.pallas_call(
      _head_body,
      grid=grid,
      in_specs=[
          pl.BlockSpec((_ROW_BLK, D), lambda i: (i, 0)),
          pl.BlockSpec((D, D), lambda i: (0, 0)),
          pl.BlockSpec((1, D), lambda i: (0, 0)),
      ],
      out_specs=pl.BlockSpec((_ROW_BLK, D), lambda i: (i, 0)),
      out_shape=jax.ShapeDtypeStruct((N_PAD, D), F32),
  )(h, wcat, bvec)


def kernel(x, edge_index, w, W1, b1, W2, b2, Wc, bc):
  n, d = x.shape
  e = w.shape[0]
  src = edge_index[0].astype(I32)
  dst = edge_index[1].astype(I32)

  grain = NW * CHUNK * NBUF
  e_pad = ((e + grain - 1) // grain) * grain
  cpw = e_pad // (NW * CHUNK)
  pads = e_pad - e
  pad_idx = (n + (jnp.arange(pads, dtype=I32) % (N_PAD - n))).astype(I32)
  src_p = jnp.concatenate([src, pad_idx])
  dst_p = jnp.concatenate([dst, pad_idx])
  src3 = src_p.reshape(NW, cpw, CHUNK)
  dst3 = dst_p.reshape(NW, cpw, CHUNK)
  pk3 = (src_p | (dst_p << PKBITS)).reshape(NW, cpw, CHUNK)
  w3 = jnp.concatenate([w, jnp.zeros((pads,), F32)]).reshape(NW, cpw, CHUNK)

  x_pad = jnp.pad(x, ((0, N_PAD - n), (0, 0)))

  deg2 = _sc_deg(src3, w3, cpw)
  disc = _tc_dis(deg2).reshape(N_PAD, 1)
  xs = _tc_scale(x_pad, disc)

  P1 = _sc_spmv(xs, pk3, w3, cpw)
  t1, t1s = _tc_psum2(P1, disc)
  Q1 = _sc_spmv(t1s, pk3, w3, cpw)
  h1, h1s = _tc_layer(x_pad, t1, Q1, disc, W1, b1, relu=True)

  P2 = _sc_spmv(h1s, pk3, w3, cpw)
  t2, t2s = _tc_psum2(P2, disc)
  Q2 = _sc_spmv(t2s, pk3, w3, cpw)
  h2, _ = _tc_layer(h1, t2, Q2, disc, W2, b2, relu=False)

  wcat = jnp.zeros((D, D), F32).at[:, 0:2].set(Wc[:d]).at[:, 2:4].set(Wc[d:])
  bvec = jnp.zeros((1, D), F32).at[0, 2:4].set(bc)
  t4 = _tc_head(h2, wcat, bvec)[:, :4]
  o0, o1 = _sc_edge_out(t4, src3, dst3, cpw)
  return jnp.stack([o0.reshape(-1)[:e], o1.reshape(-1)[:e]], axis=-1)


# speedup vs baseline: 17.3357x; 1.2928x over previous
import functools
import jax
import jax.numpy as jnp
from jax import lax
from jax.experimental import pallas as pl
from jax.experimental.pallas import tpu as pltpu
from jax.experimental.pallas import tpu_sc as plsc

F32 = jnp.float32
I32 = jnp.int32

NC = 2
NS = 16
NW = NC * NS
LANES = 16
CHUNK = 64
N = 10000
N_PAD = 10240
D = 128
ROWS_PER_TILE = N_PAD // NS
NBUF = 4
PKBITS = 14

_params = pltpu.CompilerParams(
    needs_layout_passes=False, use_tc_tiling_on_sc=False)


def _mesh():
  return plsc.VectorSubcoreMesh(
      core_axis_name="c", subcore_axis_name="s", num_cores=NC, num_subcores=NS)


def _worker_ids():
  c = lax.axis_index("c")
  s = lax.axis_index("s")
  return c, s, c * NS + s


def _zero_vmem_1d(ref, nwords):
  def zloop(i, _):
    ref[pl.ds(i * LANES, LANES)] = jnp.zeros((LANES,), F32)
    return 0
  lax.fori_loop(0, nwords // LANES, zloop, 0)


def _deg_body(cpw, src3_hbm, w3_hbm, out_hbm, src_all, w_all, zbuf, acc_sh,
              sem0, sem1):
  c, s, w = _worker_ids()
  _zero_vmem_1d(zbuf, CHUNK)
  for r in range(ROWS_PER_TILE // CHUNK):
    pltpu.sync_copy(zbuf, acc_sh.at[pl.ds(s * ROWS_PER_TILE + r * CHUNK, CHUNK)])
  pltpu.sync_copy(src3_hbm.at[w], src_all)
  pltpu.sync_copy(w3_hbm.at[w], w_all)
  plsc.subcore_barrier()

  sems = (sem0, sem1)
  waves = cpw // LANES
  for wave in range(waves):
    for i in range(LANES):
      j = wave * LANES + i
      pltpu.async_copy(w_all.at[j], acc_sh.at[src_all.at[j]], sems[wave % 2],
                       add=True)
    if wave > 0:
      for i in range(LANES):
        jj = (wave - 1) * LANES + i
        pltpu.make_async_copy(w_all.at[jj], acc_sh.at[src_all.at[jj]],
                              sems[(wave - 1) % 2]).wait()
  for i in range(LANES):
    jj = (waves - 1) * LANES + i
    pltpu.make_async_copy(w_all.at[jj], acc_sh.at[src_all.at[jj]],
                          sems[(waves - 1) % 2]).wait()
  plsc.subcore_barrier()
  pltpu.sync_copy(acc_sh.at[pl.ds(s * ROWS_PER_TILE, ROWS_PER_TILE)],
                  out_hbm.at[c, pl.ds(s * ROWS_PER_TILE, ROWS_PER_TILE)])


def _sc_deg(src3, w3, cpw):
  k = pl.kernel(
      functools.partial(_deg_body, cpw),
      out_type=jax.ShapeDtypeStruct((NC, N_PAD), F32),
      mesh=_mesh(),
      compiler_params=_params,
      scratch_types=[
          pltpu.VMEM((cpw, CHUNK), I32),
          pltpu.VMEM((cpw, CHUNK), F32),
          pltpu.VMEM((CHUNK,), F32),
          pltpu.VMEM_SHARED((N_PAD,), F32),
          pltpu.SemaphoreType.DMA,
          pltpu.SemaphoreType.DMA,
      ],
  )
  return k(src3, w3)


def _spmv_body(cpw, tab_hbm, pk3_hbm, w3_hbm, out_hbm,
               pk_all, srcb, dstb, wb, rows, acc_sh, gsems, wsems):
  c, s, w = _worker_ids()
  def zrow(i, _):
    for cc in range(D // LANES):
      rows[0][i, pl.ds(cc * LANES, LANES)] = jnp.zeros((LANES,), F32)
    return 0
  lax.fori_loop(0, CHUNK, zrow, 0)
  for r in range(ROWS_PER_TILE // CHUNK):
    pltpu.sync_copy(rows[0], acc_sh.at[pl.ds(s * ROWS_PER_TILE + r * CHUNK, CHUNK)])
  pltpu.sync_copy(pk3_hbm.at[w], pk_all)
  plsc.subcore_barrier()

  mask = jnp.full((LANES,), (1 << PKBITS) - 1, I32)
  shift = jnp.full((LANES,), PKBITS, I32)

  def unpack(j, b):
    for g in range(CHUNK // LANES):
      sl = pl.ds(g * LANES, LANES)
      p = pk_all[j, sl]
      srcb[b][sl] = p & mask
      dstb[b][sl] = lax.shift_right_logical(p, shift)

  def issue_gather(j, b):
    pltpu.async_copy(tab_hbm.at[srcb[b]], rows[b], gsems[b])
    pltpu.async_copy(w3_hbm.at[w, j], wb[b], gsems[b])

  def wait_gather(j, b):
    pltpu.make_async_copy(tab_hbm.at[srcb[b]], rows[b], gsems[b]).wait()
    pltpu.make_async_copy(w3_hbm.at[w, j], wb[b], gsems[b]).wait()

  for p in range(2):
    unpack(p, p)
    issue_gather(p, p)

  def body(j0, _):
    for b in range(NBUF):
      j = j0 * NBUF + b
      bn = (b + 2) % NBUF

      @pl.when(j + 2 < cpw)
      def _():
        @pl.when(j >= 2)
        def _():
          pltpu.make_async_copy(rows[bn], acc_sh.at[dstb[bn]],
                                wsems[bn]).wait()
        unpack(j + 2, bn)
        issue_gather(j + 2, bn)

      wait_gather(j, b)
      def scale(e, _):
        sp = plsc.load_gather(wb[b], [jnp.full((LANES,), e, I32)])
        for cc in range(D // LANES):
          sl = pl.ds(cc * LANES, LANES)
          rows[b][e, sl] = rows[b][e, sl] * sp
        return 0
      lax.fori_loop(0, CHUNK, scale, 0)
      pltpu.async_copy(rows[b], acc_sh.at[dstb[b]], wsems[b], add=True)
    return 0
  lax.fori_loop(0, cpw // NBUF, body, 0)
  for b in range(NBUF):
    pltpu.make_async_copy(rows[b], acc_sh.at[dstb[b]], wsems[b]).wait()
  plsc.subcore_barrier()
  pltpu.sync_copy(acc_sh.at[pl.ds(s * ROWS_PER_TILE, ROWS_PER_TILE)],
                  out_hbm.at[c, pl.ds(s * ROWS_PER_TILE, ROWS_PER_TILE)])


def _sc_spmv(tab, pk3, w3, cpw):
  k = pl.kernel(
      functools.partial(_spmv_body, cpw),
      out_type=jax.ShapeDtypeStruct((NC, N_PAD, D), F32),
      mesh=_mesh(),
      compiler_params=_params,
      scratch_types=[
          pltpu.VMEM((cpw, CHUNK), I32),
          [pltpu.VMEM((CHUNK,), I32) for _ in range(NBUF)],
          [pltpu.VMEM((CHUNK,), I32) for _ in range(NBUF)],
          [pltpu.VMEM((CHUNK,), F32) for _ in range(NBUF)],
          [pltpu.VMEM((CHUNK, D), F32) for _ in range(NBUF)],
          pltpu.VMEM_SHARED((N_PAD, D), F32),
          [pltpu.SemaphoreType.DMA for _ in range(NBUF)],
          [pltpu.SemaphoreType.DMA for _ in range(NBUF)],
      ],
  )
  return k(tab, pk3, w3)


def _edge_body(cpw, t4_hbm, src3_hbm, dst3_hbm, o0_hbm, o1_hbm,
               t4_v, src_all, dst_all, o0_all, o1_all):
  c, s, w = _worker_ids()
  pltpu.sync_copy(t4_hbm, t4_v)
  pltpu.sync_copy(src3_hbm.at[w], src_all)
  pltpu.sync_copy(dst3_hbm.at[w], dst_all)
  k0 = jnp.zeros((LANES,), I32)
  k1 = jnp.full((LANES,), 1, I32)
  k2 = jnp.full((LANES,), 2, I32)
  k3 = jnp.full((LANES,), 3, I32)

  def chunk(j, _):
    for g in range(CHUNK // LANES):
      sl = pl.ds(g * LANES, LANES)
      isrc = src_all[j, sl]
      idst = dst_all[j, sl]
      o0_all[j, sl] = (plsc.load_gather(t4_v, [isrc, k0]) +
                       plsc.load_gather(t4_v, [idst, k2]))
      o1_all[j, sl] = (plsc.load_gather(t4_v, [isrc, k1]) +
                       plsc.load_gather(t4_v, [idst, k3]))
    return 0
  lax.fori_loop(0, cpw, chunk, 0)
  pltpu.sync_copy(o0_all, o0_hbm.at[w])
  pltpu.sync_copy(o1_all, o1_hbm.at[w])


def _sc_edge_out(t4, src3, dst3, cpw):
  k = pl.kernel(
      functools.partial(_edge_body, cpw),
      out_type=(jax.ShapeDtypeStruct((NW, cpw, CHUNK), F32),
                jax.ShapeDtypeStruct((NW, cpw, CHUNK), F32)),
      mesh=_mesh(),
      compiler_params=_params,
      scratch_types=[
          pltpu.VMEM((N_PAD, 4), F32),
          pltpu.VMEM((cpw, CHUNK), I32),
          pltpu.VMEM((cpw, CHUNK), I32),
          pltpu.VMEM((cpw, CHUNK), F32),
          pltpu.VMEM((cpw, CHUNK), F32),
      ],
  )
  return k(t4, src3, dst3)


def _dis_kernel_body(deg_ref, out_ref):
  d = deg_ref[0] + deg_ref[1]
  out_ref[...] = jnp.where(d > 0, lax.rsqrt(jnp.where(d > 0, d, 1.0)), 0.0)


def _tc_dis(deg2):
  deg2 = deg2.reshape(NC, N_PAD // D, D)
  return pl.pallas_call(
      _dis_kernel_body,
      out_shape=jax.ShapeDtypeStruct((N_PAD // D, D), F32),
  )(deg2)


_ROW_BLK = 1024


def _scale_body(x_ref, d_ref, o_ref):
  o_ref[...] = x_ref[...] * d_ref[...]


def _tc_scale(x_pad, disc):
  grid = (N_PAD // _ROW_BLK,)
  return pl.pallas_call(
      _scale_body,
      grid=grid,
      in_specs=[
          pl.BlockSpec((_ROW_BLK, D), lambda i: (i, 0)),
          pl.BlockSpec((_ROW_BLK, 1), lambda i: (i, 0)),
      ],
      out_specs=pl.BlockSpec((_ROW_BLK, D), lambda i: (i, 0)),
      out_shape=jax.ShapeDtypeStruct((N_PAD, D), F32),
  )(x_pad, disc)


def _psum2_body(p_ref, d_ref, t_ref, ts_ref):
  dcol = d_ref[...]
  t = -(p_ref[0] + p_ref[1]) * dcol
  t_ref[...] = t
  ts_ref[...] = t * dcol


def _tc_psum2(P, disc):
  grid = (N_PAD // _ROW_BLK,)
  return pl.pallas_call(
      _psum2_body,
      grid=grid,
      in_specs=[
          pl.BlockSpec((NC, _ROW_BLK, D), lambda i: (0, i, 0)),
          pl.BlockSpec((_ROW_BLK, 1), lambda i: (i, 0)),
      ],
      out_specs=(pl.BlockSpec((_ROW_BLK, D), lambda i: (i, 0)),
                 pl.BlockSpec((_ROW_BLK, D), lambda i: (i, 0))),
      out_shape=(jax.ShapeDtypeStruct((N_PAD, D), F32),
                 jax.ShapeDtypeStruct((N_PAD, D), F32)),
  )(P, disc)


def _layer_body(relu, x_ref, t1_ref, q_ref, d_ref, w_ref, b_ref,
                o_ref, os_ref):
  x = x_ref[...]
  t1 = t1_ref[...]
  dcol = d_ref[...]
  t2 = -2.0 * (q_ref[0] + q_ref[1]) * dcol - x
  o = (jnp.dot(x, w_ref[0], preferred_element_type=F32) +
       jnp.dot(t1, w_ref[1], preferred_element_type=F32) +
       jnp.dot(t2, w_ref[2], preferred_element_type=F32) + b_ref[...])
  if relu:
    o = jnp.maximum(o, 0.0)
  o_ref[...] = o
  os_ref[...] = o * dcol


def _tc_layer(x_pad, t1, Q, disc, W, b, relu):
  grid = (N_PAD // _ROW_BLK,)
  return pl.pallas_call(
      functools.partial(_layer_body, relu),
      grid=grid,
      in_specs=[
          pl.BlockSpec((_ROW_BLK, D), lambda i: (i, 0)),
          pl.BlockSpec((_ROW_BLK, D), lambda i: (i, 0)),
          pl.BlockSpec((NC, _ROW_BLK, D), lambda i: (0, i, 0)),
          pl.BlockSpec((_ROW_BLK, 1), lambda i: (i, 0)),
          pl.BlockSpec((3, D, D), lambda i: (0, 0, 0)),
          pl.BlockSpec((1, D), lambda i: (0, 0)),
      ],
      out_specs=(pl.BlockSpec((_ROW_BLK, D), lambda i: (i, 0)),
                 pl.BlockSpec((_ROW_BLK, D), lambda i: (i, 0))),
      out_shape=(jax.ShapeDtypeStruct((N_PAD, D), F32),
                 jax.ShapeDtypeStruct((N_PAD, D), F32)),
  )(x_pad, t1, Q, disc, W, b.reshape(1, D))


def _head_body(h_ref, w_ref, b_ref, o_ref):
  o_ref[...] = (jnp.dot(h_ref[...], w_ref[...], preferred_element_type=F32)
                + b_ref[...])


def _tc_head(h, wcat, bvec):
  grid = (N_PAD // _ROW_BLK,)
  return pl.pallas_call(
      _head_body,
      grid=grid,
      in_specs=[
          pl.BlockSpec((_ROW_BLK, D), lambda i: (i, 0)),
          pl.BlockSpec((D, D), lambda i: (0, 0)),
          pl.BlockSpec((1, D), lambda i: (0, 0)),
      ],
      out_specs=pl.BlockSpec((_ROW_BLK, D), lambda i: (i, 0)),
      out_shape=jax.ShapeDtypeStruct((N_PAD, D), F32),
  )(h, wcat, bvec)


def kernel(x, edge_index, w, W1, b1, W2, b2, Wc, bc):
  n, d = x.shape
  e = w.shape[0]
  src = edge_index[0].astype(I32)
  dst = edge_index[1].astype(I32)

  grain = NW * CHUNK * NBUF
  e_pad = ((e + grain - 1) // grain) * grain
  cpw = e_pad // (NW * CHUNK)
  pads = e_pad - e
  pad_idx = (n + (jnp.arange(pads, dtype=I32) % (N_PAD - n))).astype(I32)
  src_p = jnp.concatenate([src, pad_idx])
  dst_p = jnp.concatenate([dst, pad_idx])
  src3 = src_p.reshape(NW, cpw, CHUNK)
  dst3 = dst_p.reshape(NW, cpw, CHUNK)
  pk3 = (src_p | (dst_p << PKBITS)).reshape(NW, cpw, CHUNK)
  w3 = jnp.concatenate([w, jnp.zeros((pads,), F32)]).reshape(NW, cpw, CHUNK)

  x_pad = jnp.pad(x, ((0, N_PAD - n), (0, 0)))

  deg2 = _sc_deg(src3, w3, cpw)
  disc = _tc_dis(deg2).reshape(N_PAD, 1)
  xs = _tc_scale(x_pad, disc)

  P1 = _sc_spmv(xs, pk3, w3, cpw)
  t1, t1s = _tc_psum2(P1, disc)
  Q1 = _sc_spmv(t1s, pk3, w3, cpw)
  h1, h1s = _tc_layer(x_pad, t1, Q1, disc, W1, b1, relu=True)

  P2 = _sc_spmv(h1s, pk3, w3, cpw)
  t2, t2s = _tc_psum2(P2, disc)
  Q2 = _sc_spmv(t2s, pk3, w3, cpw)
  h2, _ = _tc_layer(h1, t2, Q2, disc, W2, b2, relu=False)

  wcat = jnp.zeros((D, D), F32).at[:, 0:2].set(Wc[:d]).at[:, 2:4].set(Wc[d:])
  bvec = jnp.zeros((1, D), F32).at[0, 2:4].set(bc)
  t4 = _tc_head(h2, wcat, bvec)[:, :4]
  o0, o1 = _sc_edge_out(t4, src3, dst3, cpw)
  return jnp.stack([o0.reshape(-1)[:e], o1.reshape(-1)[:e]], axis=-1)
